# baseline (speedup 1.0000x reference)
import functools

import jax
import jax.numpy as jnp
from jax import lax
from jax.experimental import pallas as pl
from jax.experimental.pallas import tpu as pltpu
from jax.experimental.pallas import tpu_sc as plsc

_NUM_TILES = 32


def _make_kernel():
    mesh = plsc.VectorSubcoreMesh(
        core_axis_name="c", subcore_axis_name="s", num_cores=2, num_subcores=16
    )

    @functools.partial(
        pl.kernel,
        out_type=[
            jax.ShapeDtypeStruct((8, 3, 4, 8, 8, 128), jnp.float32),
            jax.ShapeDtypeStruct((3, 8, 8, 128), jnp.float32),
        ],
        mesh=mesh,
        scratch_types=[
            pltpu.VMEM((3, 128, 128), jnp.float32),
            pltpu.VMEM((32, 256), jnp.int32),
            pltpu.VMEM((2, 128), jnp.int32),
            pltpu.VMEM((24, 8, 128), jnp.float32),
            pltpu.VMEM((6, 128), jnp.float32),
            pltpu.SemaphoreType.DMA,
            pltpu.SemaphoreType.DMA,
            pltpu.SemaphoreType.DMA,
            pltpu.SemaphoreType.DMA,
            pltpu.SemaphoreType.DMA,
        ],
        compiler_params=pltpu.CompilerParams(
            use_tc_tiling_on_sc=False, needs_layout_passes=False
        ),
    )
    def k(pts6, idxt_hbm, cidx3_hbm, out6, cout6,
          table_v, idx_v, cidx_v, out_v, cout_v,
          isem, p0sem, p1sem, p2sem, osem):
        tid = lax.axis_index("s") * 2 + lax.axis_index("c")
        b = tid // 4
        q = tid % 4

        psems = [p0sem, p1sem, p2sem]
        pcopies = [
            pltpu.async_copy(pts6.at[c, :, b, :], table_v.at[c], psems[c])
            for c in range(3)
        ]
        icopies = [
            pltpu.async_copy(idxt_hbm.at[b, :, pl.ds(q * 256, 256)], idx_v, isem),
            pltpu.async_copy(cidx3_hbm.at[pl.ds(q * 2, 2), b, :], cidx_v, isem),
        ]
        for cp in icopies:
            cp.wait()

        for c in range(3):
            pcopies[c].wait()
            cc = jnp.full((16,), c, jnp.int32)

            def gbody(ptp, carry, c=c, cc=cc):
                def wbody(w, carry2, c=c, cc=cc, ptp=ptp):
                    cn = cidx_v[ptp, pl.ds(w * 16, 16)]
                    vc = plsc.load_gather(table_v, [cc, cn >> 7, cn & 127])
                    cout_v[c * 2 + ptp, pl.ds(w * 16, 16)] = vc
                    pbase = ptp * 128 + w * 16
                    for kg in range(4):
                        chunk = (c * 4 + kg) * 2 + ptp
                        for ks in range(8):
                            ni = idx_v[kg * 8 + ks, pl.ds(pbase, 16)]
                            vn = plsc.load_gather(table_v, [cc, ni >> 7, ni & 127])
                            out_v[chunk, ks, pl.ds(w * 16, 16)] = vn - vc
                    return carry2

                lax.fori_loop(0, 8, wbody, carry)
                for kg in range(4):
                    pltpu.async_copy(
                        out_v.at[(c * 4 + kg) * 2 + ptp],
                        out6.at[b, c, kg, q * 2 + ptp], osem)
                pltpu.async_copy(
                    cout_v.at[c * 2 + ptp], cout6.at[c, q * 2 + ptp, b], osem)
                return carry

            lax.fori_loop(0, 2, gbody, 0)

        def drain_chunks(i, carry):
            pltpu.make_async_copy(out_v.at[0], out6.at[0, 0, 0, 0], osem).wait()
            return carry

        lax.fori_loop(0, 24, drain_chunks, 0)

        def drain_centers(i, carry):
            pltpu.make_async_copy(cout_v.at[0], cout6.at[0, 0, 0], osem).wait()
            return carry

        lax.fori_loop(0, 6, drain_centers, 0)

    return k


def kernel(points, patches_idx0, centers_idx0):
    B, N, _ = points.shape
    _, P, K = patches_idx0.shape
    pts6 = points.transpose(2, 0, 1).reshape(3, 8, 128, 128).transpose(0, 2, 1, 3)
    cidx3 = centers_idx0.astype(jnp.int32).reshape(8, 8, 128).transpose(1, 0, 2)
    out6, cout6 = _make_kernel()(
        pts6,
        patches_idx0.astype(jnp.int32).transpose(0, 2, 1),
        cidx3,
    )
    patches = out6.transpose(0, 3, 5, 2, 4, 1).reshape(B, P, K, 3)
    centers = cout6.transpose(2, 1, 3, 0).reshape(B, P, 3)
    return patches, centers


# speedup vs baseline: 1.0981x; 1.0981x over previous
import functools

import jax
import jax.numpy as jnp
from jax import lax
from jax.experimental import pallas as pl
from jax.experimental.pallas import tpu as pltpu
from jax.experimental.pallas import tpu_sc as plsc

_NUM_TILES = 32


def _make_kernel():
    mesh = plsc.VectorSubcoreMesh(
        core_axis_name="c", subcore_axis_name="s", num_cores=2, num_subcores=16
    )

    @functools.partial(
        pl.kernel,
        out_type=[
            jax.ShapeDtypeStruct((8, 3, 4, 8, 8, 128), jnp.float32),
            jax.ShapeDtypeStruct((3, 8, 8, 128), jnp.float32),
        ],
        mesh=mesh,
        scratch_types=[
            pltpu.VMEM((3, 128, 128), jnp.float32),
            pltpu.VMEM((32, 256), jnp.int32),
            pltpu.VMEM((2, 128), jnp.int32),
            pltpu.VMEM((24, 8, 128), jnp.float32),
            pltpu.VMEM((6, 128), jnp.float32),
            pltpu.SemaphoreType.DMA,
            pltpu.SemaphoreType.DMA,
            pltpu.SemaphoreType.DMA,
            pltpu.SemaphoreType.DMA,
            pltpu.SemaphoreType.DMA,
        ],
        compiler_params=pltpu.CompilerParams(
            use_tc_tiling_on_sc=False, needs_layout_passes=False
        ),
    )
    def k(pts6, idxt_hbm, cidx3_hbm, out6, cout6,
          table_v, idx_v, cidx_v, out_v, cout_v,
          isem, p0sem, p1sem, p2sem, osem):
        tid = lax.axis_index("s") * 2 + lax.axis_index("c")
        b = tid // 4
        q = tid % 4

        psems = [p0sem, p1sem, p2sem]
        pcopies = [
            pltpu.async_copy(pts6.at[c, :, b, :], table_v.at[c], psems[c])
            for c in range(3)
        ]
        icopies = [
            pltpu.async_copy(idxt_hbm.at[b, :, pl.ds(q * 256, 256)], idx_v, isem),
            pltpu.async_copy(cidx3_hbm.at[pl.ds(q * 2, 2), b, :], cidx_v, isem),
        ]
        for cp in icopies:
            cp.wait()

        outcopies = []
        for c in range(3):
            pcopies[c].wait()
            cc = jnp.full((16,), c, jnp.int32)
            for ptp in range(2):
                def wbody(w, carry, c=c, cc=cc, ptp=ptp):
                    cn = cidx_v[ptp, pl.ds(w * 16, 16)]
                    vc = plsc.load_gather(table_v, [cc, cn >> 7, cn & 127])
                    cout_v[c * 2 + ptp, pl.ds(w * 16, 16)] = vc
                    pbase = ptp * 128 + w * 16
                    for kg in range(4):
                        chunk = (c * 4 + kg) * 2 + ptp
                        for ks in range(8):
                            ni = idx_v[kg * 8 + ks, pl.ds(pbase, 16)]
                            vn = plsc.load_gather(table_v, [cc, ni >> 7, ni & 127])
                            out_v[chunk, ks, pl.ds(w * 16, 16)] = vn - vc
                    return carry

                lax.fori_loop(0, 8, wbody, 0)

                for kg in range(4):
                    outcopies.append(pltpu.async_copy(
                        out_v.at[(c * 4 + kg) * 2 + ptp],
                        out6.at[b, c, kg, q * 2 + ptp], osem))
                outcopies.append(pltpu.async_copy(
                    cout_v.at[c * 2 + ptp], cout6.at[c, q * 2 + ptp, b], osem))
        for cp in outcopies:
            cp.wait()

    return k


def kernel(points, patches_idx0, centers_idx0):
    B, N, _ = points.shape
    _, P, K = patches_idx0.shape
    pts6 = points.transpose(2, 0, 1).reshape(3, 8, 128, 128).transpose(0, 2, 1, 3)
    cidx3 = centers_idx0.astype(jnp.int32).reshape(8, 8, 128).transpose(1, 0, 2)
    out6, cout6 = _make_kernel()(
        pts6,
        patches_idx0.astype(jnp.int32).transpose(0, 2, 1),
        cidx3,
    )
    patches = out6.transpose(0, 3, 5, 2, 4, 1).reshape(B, P, K, 3)
    centers = cout6.transpose(2, 1, 3, 0).reshape(B, P, 3)
    return patches, centers


# speedup vs baseline: 1.2218x; 1.1126x over previous
import functools

import jax
import jax.numpy as jnp
from jax import lax
from jax.experimental import pallas as pl
from jax.experimental.pallas import tpu as pltpu
from jax.experimental.pallas import tpu_sc as plsc

_NUM_TILES = 32


def _make_kernel():
    mesh = plsc.VectorSubcoreMesh(
        core_axis_name="c", subcore_axis_name="s", num_cores=2, num_subcores=16
    )

    @functools.partial(
        pl.kernel,
        out_type=[
            jax.ShapeDtypeStruct((8, 3, 4, 8, 8, 128), jnp.float32),
            jax.ShapeDtypeStruct((3, 8, 8, 128), jnp.float32),
        ],
        mesh=mesh,
        scratch_types=[
            pltpu.VMEM((3, 128, 128), jnp.float32),
            pltpu.VMEM((32, 256), jnp.int32),
            pltpu.VMEM((2, 128), jnp.int32),
            pltpu.VMEM((24, 8, 128), jnp.float32),
            pltpu.VMEM((6, 128), jnp.float32),
            pltpu.SemaphoreType.DMA,
            pltpu.SemaphoreType.DMA,
            pltpu.SemaphoreType.DMA,
            pltpu.SemaphoreType.DMA,
            pltpu.SemaphoreType.DMA,
        ],
        compiler_params=pltpu.CompilerParams(
            use_tc_tiling_on_sc=False, needs_layout_passes=False
        ),
    )
    def k(pts6, idxt_hbm, cidx3_hbm, out6, cout6,
          table_v, idx_v, cidx_v, out_v, cout_v,
          isem, p0sem, p1sem, p2sem, osem):
        tid = lax.axis_index("s") * 2 + lax.axis_index("c")
        b = tid // 4
        q = tid % 4

        psems = [p0sem, p1sem, p2sem]
        pcopies = [
            pltpu.async_copy(pts6.at[c, :, b, :], table_v.at[c], psems[c])
            for c in range(3)
        ]
        icopies = [
            pltpu.async_copy(idxt_hbm.at[b, :, pl.ds(q * 256, 256)], idx_v, isem),
            pltpu.async_copy(cidx3_hbm.at[pl.ds(q * 2, 2), b, :], cidx_v, isem),
        ]
        for cp in icopies:
            cp.wait()

        for cp in pcopies:
            cp.wait()
        ccs = [jnp.full((16,), c, jnp.int32) for c in range(3)]

        outcopies = []
        for ptp in range(2):
            def wbody(w, carry, ptp=ptp):
                cn = cidx_v[ptp, pl.ds(w * 16, 16)]
                chi = cn >> 7
                clo = cn & 127
                vcs = [plsc.load_gather(table_v, [ccs[c], chi, clo])
                       for c in range(3)]
                for c in range(3):
                    cout_v[c * 2 + ptp, pl.ds(w * 16, 16)] = vcs[c]
                pbase = ptp * 128 + w * 16
                for kg in range(4):
                    for ks in range(8):
                        ni = idx_v[kg * 8 + ks, pl.ds(pbase, 16)]
                        hi = ni >> 7
                        lo = ni & 127
                        for c in range(3):
                            vn = plsc.load_gather(table_v, [ccs[c], hi, lo])
                            chunk = (c * 4 + kg) * 2 + ptp
                            out_v[chunk, ks, pl.ds(w * 16, 16)] = vn - vcs[c]
                return carry

            lax.fori_loop(0, 8, wbody, 0)

            for c in range(3):
                for kg in range(4):
                    outcopies.append(pltpu.async_copy(
                        out_v.at[(c * 4 + kg) * 2 + ptp],
                        out6.at[b, c, kg, q * 2 + ptp], osem))
                outcopies.append(pltpu.async_copy(
                    cout_v.at[c * 2 + ptp], cout6.at[c, q * 2 + ptp, b], osem))
        for cp in outcopies:
            cp.wait()

    return k


def kernel(points, patches_idx0, centers_idx0):
    B, N, _ = points.shape
    _, P, K = patches_idx0.shape
    pts6 = points.transpose(2, 0, 1).reshape(3, 8, 128, 128).transpose(0, 2, 1, 3)
    cidx3 = centers_idx0.astype(jnp.int32).reshape(8, 8, 128).transpose(1, 0, 2)
    out6, cout6 = _make_kernel()(
        pts6,
        patches_idx0.astype(jnp.int32).transpose(0, 2, 1),
        cidx3,
    )
    patches = out6.transpose(0, 3, 5, 2, 4, 1).reshape(B, P, K, 3)
    centers = cout6.transpose(2, 1, 3, 0).reshape(B, P, 3)
    return patches, centers


# speedup vs baseline: 1.6351x; 1.3383x over previous
import functools

import jax
import jax.numpy as jnp
from jax import lax
from jax.experimental import pallas as pl
from jax.experimental.pallas import tpu as pltpu
from jax.experimental.pallas import tpu_sc as plsc

_NUM_TILES = 32


def _make_kernel():
    mesh = plsc.VectorSubcoreMesh(
        core_axis_name="c", subcore_axis_name="s", num_cores=2, num_subcores=16
    )

    @functools.partial(
        pl.kernel,
        out_type=[
            jax.ShapeDtypeStruct((8, 3, 4, 8, 8, 128), jnp.float32),
            jax.ShapeDtypeStruct((3, 8, 8, 128), jnp.float32),
        ],
        mesh=mesh,
        scratch_types=[
            pltpu.VMEM((3, 128, 128), jnp.float32),
            pltpu.VMEM((32, 256), jnp.int32),
            pltpu.VMEM((2, 128), jnp.int32),
            pltpu.VMEM((24, 8, 128), jnp.float32),
            pltpu.VMEM((6, 128), jnp.float32),
            pltpu.SemaphoreType.DMA,
            pltpu.SemaphoreType.DMA,
            pltpu.SemaphoreType.DMA,
            pltpu.SemaphoreType.DMA,
            pltpu.SemaphoreType.DMA,
        ],
        compiler_params=pltpu.CompilerParams(
            use_tc_tiling_on_sc=False, needs_layout_passes=False
        ),
    )
    def k(pts6, idxt_hbm, cidx3_hbm, out6, cout6,
          table_v, idx_v, cidx_v, out_v, cout_v,
          isem, p0sem, p1sem, p2sem, osem):
        tid = lax.axis_index("s") * 2 + lax.axis_index("c")
        b = tid // 4
        q = tid % 4

        psems = [p0sem, p1sem, p2sem]
        pcopies = [
            pltpu.async_copy(pts6.at[c, :, b, :], table_v.at[c], psems[c])
            for c in range(3)
        ]
        icopies = [
            pltpu.async_copy(idxt_hbm.at[b, :, pl.ds(q * 256, 256)], idx_v, isem),
            pltpu.async_copy(cidx3_hbm.at[pl.ds(q * 2, 2), b, :], cidx_v, isem),
        ]
        for cp in icopies:
            cp.wait()

        for cp in pcopies:
            cp.wait()
        ccs = [jnp.full((16,), c, jnp.int32) for c in range(3)]

        outcopies = []
        for ptp in range(2):
            @functools.partial(plsc.parallel_loop, 0, 8)
            def wbody(w, ptp=ptp):
                cn = cidx_v[ptp, pl.ds(w * 16, 16)]
                chi = cn >> 7
                clo = cn & 127
                vcs = [plsc.load_gather(table_v, [ccs[c], chi, clo])
                       for c in range(3)]
                for c in range(3):
                    cout_v[c * 2 + ptp, pl.ds(w * 16, 16)] = vcs[c]
                pbase = ptp * 128 + w * 16
                for kg in range(4):
                    for ks in range(8):
                        ni = idx_v[kg * 8 + ks, pl.ds(pbase, 16)]
                        hi = ni >> 7
                        lo = ni & 127
                        for c in range(3):
                            vn = plsc.load_gather(table_v, [ccs[c], hi, lo])
                            chunk = (c * 4 + kg) * 2 + ptp
                            out_v[chunk, ks, pl.ds(w * 16, 16)] = vn - vcs[c]

            for c in range(3):
                for kg in range(4):
                    outcopies.append(pltpu.async_copy(
                        out_v.at[(c * 4 + kg) * 2 + ptp],
                        out6.at[b, c, kg, q * 2 + ptp], osem))
                outcopies.append(pltpu.async_copy(
                    cout_v.at[c * 2 + ptp], cout6.at[c, q * 2 + ptp, b], osem))
        for cp in outcopies:
            cp.wait()

    return k


def kernel(points, patches_idx0, centers_idx0):
    B, N, _ = points.shape
    _, P, K = patches_idx0.shape
    pts6 = points.transpose(2, 0, 1).reshape(3, 8, 128, 128).transpose(0, 2, 1, 3)
    cidx3 = centers_idx0.astype(jnp.int32).reshape(8, 8, 128).transpose(1, 0, 2)
    out6, cout6 = _make_kernel()(
        pts6,
        patches_idx0.astype(jnp.int32).transpose(0, 2, 1),
        cidx3,
    )
    patches = out6.transpose(0, 3, 5, 2, 4, 1).reshape(B, P, K, 3)
    centers = cout6.transpose(2, 1, 3, 0).reshape(B, P, 3)
    return patches, centers
